# baseline (speedup 1.0000x reference)
import functools

import jax
import jax.numpy as jnp
from jax import lax
from jax.experimental import pallas as pl
from jax.experimental.pallas import tpu as pltpu
from jax.experimental.pallas import tpu_sc as plsc

NC = 2
NS = 16
CK = 128
SPLIT = 96
GG = 18
NG = 9
NR = 3


def _sc_segment_accumulate(tbl, idx, zro, *, r_acc, d_t):
    rows_main = 8 * (-(-r_acc // (NS * 8)))
    rows_last = r_acc - (NS - 1) * rows_main
    assert rows_last > 0
    mesh = plsc.VectorSubcoreMesh(core_axis_name="c", subcore_axis_name="s")

    @functools.partial(
        pl.kernel,
        out_type=jax.ShapeDtypeStruct((NC, r_acc, d_t), jnp.float32),
        mesh=mesh,
        scratch_types=[
            pltpu.VMEM((GG, 2, CK), jnp.int32),
            pltpu.VMEM((GG, 2, CK), jnp.int32),
            pltpu.VMEM((CK, d_t), jnp.float32),
            pltpu.VMEM((CK, d_t), jnp.float32),
            pltpu.VMEM((CK, d_t), jnp.float32),
            pltpu.VMEM_SHARED((r_acc, d_t), jnp.float32),
            pltpu.SemaphoreType.DMA,
            pltpu.SemaphoreType.DMA,
            pltpu.SemaphoreType.DMA,
            pltpu.SemaphoreType.DMA,
            pltpu.SemaphoreType.DMA,
        ],
        compiler_params=pltpu.CompilerParams(use_tc_tiling_on_sc=False),
    )
    def body(tbl_hbm, idx_hbm, zro_hbm, acc_out, idx_a, idx_b,
             rows0, rows1, rows2, acc_sh, semi_a, semi_b, sem0, sem1, sem2):
        cid = lax.axis_index("c")
        sid = lax.axis_index("s")
        rbase = pl.multiple_of(sid * rows_main, 8)

        @pl.when(sid < NS - 1)
        def _init_main():
            pltpu.sync_copy(zro_hbm.at[pl.ds(rbase, rows_main)],
                            acc_sh.at[pl.ds(rbase, rows_main)])

        @pl.when(sid == NS - 1)
        def _init_last():
            pltpu.sync_copy(zro_hbm.at[pl.ds((NS - 1) * rows_main, rows_last)],
                            acc_sh.at[pl.ds((NS - 1) * rows_main, rows_last)])

        plsc.subcore_barrier()

        rows = (rows0, rows1, rows2)
        gsems = (sem0, sem1, sem2)
        idxs = (idx_a, idx_b)
        isems = (semi_a, semi_b)

        def start_gather(idx_v, j, b):
            pltpu.async_copy(tbl_hbm.at[cid].at[idx_v.at[j, 0]],
                             rows[b], gsems[b])

        def wait_gather(b):
            pltpu.make_async_copy(tbl_hbm.at[cid, pl.ds(0, CK)],
                                  rows[b], gsems[b]).wait()

        def scatter(idx_v, j, b):
            pltpu.sync_copy(rows[b], acc_sh.at[idx_v.at[j, 1]], add=True)

        pltpu.sync_copy(idx_hbm.at[sid, 0], idx_a)

        for p in range(NG):
            cur, nxt = idxs[p % 2], idxs[(p + 1) % 2]
            if p + 1 < NG:
                pltpu.async_copy(idx_hbm.at[sid, p + 1], nxt,
                                 isems[(p + 1) % 2])
            start_gather(cur, 0, 0)
            start_gather(cur, 1, 1)

            @pl.loop(0, GG, step=NR)
            def _step(j, cur=cur):
                start_gather(cur, j + 2, 2)
                wait_gather(0)
                scatter(cur, j, 0)

                @pl.when(j + 3 < GG)
                def _r0(cur=cur, j=j):
                    start_gather(cur, j + 3, 0)

                wait_gather(1)
                scatter(cur, j + 1, 1)

                @pl.when(j + 4 < GG)
                def _r1(cur=cur, j=j):
                    start_gather(cur, j + 4, 1)

                wait_gather(2)
                scatter(cur, j + 2, 2)

            if p + 1 < NG:
                pltpu.make_async_copy(idx_hbm.at[sid, 0], nxt,
                                      isems[(p + 1) % 2]).wait()

        plsc.subcore_barrier()

        @pl.when(sid < NS - 1)
        def _wb_main():
            pltpu.sync_copy(acc_sh.at[pl.ds(rbase, rows_main)],
                            acc_out.at[cid, pl.ds(rbase, rows_main)])

        @pl.when(sid == NS - 1)
        def _wb_last():
            pltpu.sync_copy(acc_sh.at[pl.ds((NS - 1) * rows_main, rows_last)],
                            acc_out.at[cid, pl.ds((NS - 1) * rows_main,
                                                  rows_last)])

    return body(tbl, idx, zro)


def _tc_gates(acc, x, h, c, wx, wxr, wh, whr, bias, fcw, fcb, *, n, d_in, d_h):
    d_t = acc.shape[-1]
    blk = 2000
    grid = (n // blk,)

    def body(acc_ref, x_ref, h_ref, c_ref, wx_ref, wxr_ref, wh_ref, whr_ref,
             b_ref, fcw_ref, fcb_ref, out_ref, hn_ref, cn_ref):
        s = jnp.concatenate([acc_ref[0, :, :SPLIT], acc_ref[1]], axis=-1)
        deg = jnp.maximum(s[:, d_in + d_h:d_in + d_h + 1], 1.0)
        inv = 1.0 / deg
        zx = s[:, :d_in] * inv
        zh = s[:, d_in:d_in + d_h] * inv
        pre = (jnp.dot(zx, wx_ref[...], preferred_element_type=jnp.float32)
               + jnp.dot(x_ref[...], wxr_ref[...],
                         preferred_element_type=jnp.float32)
               + jnp.dot(zh, wh_ref[...], preferred_element_type=jnp.float32)
               + jnp.dot(h_ref[...], whr_ref[...],
                         preferred_element_type=jnp.float32)
               + b_ref[...])
        ig = jax.nn.sigmoid(pre[:, :d_h])
        fg = jax.nn.sigmoid(pre[:, d_h:2 * d_h])
        tg = jnp.tanh(pre[:, 2 * d_h:3 * d_h])
        og = jax.nn.sigmoid(pre[:, 3 * d_h:])
        cn = fg * c_ref[...] + ig * tg
        hn = og * jnp.tanh(cn)
        r = jnp.maximum(hn, 0.0)
        out_ref[...] = (jnp.sum(r * fcw_ref[...], axis=1, keepdims=True)
                        + fcb_ref[0, 0])
        hn_ref[...] = hn
        cn_ref[...] = cn

    full = lambda shape: pl.BlockSpec(shape, lambda i: tuple(0 for _ in shape))
    return pl.pallas_call(
        body,
        grid=grid,
        in_specs=[
            pl.BlockSpec((NC, blk, d_t), lambda i: (0, i, 0)),
            pl.BlockSpec((blk, d_in), lambda i: (i, 0)),
            pl.BlockSpec((blk, d_h), lambda i: (i, 0)),
            pl.BlockSpec((blk, d_h), lambda i: (i, 0)),
            full((d_in, 4 * d_h)),
            full((d_in, 4 * d_h)),
            full((d_h, 4 * d_h)),
            full((d_h, 4 * d_h)),
            full((1, 4 * d_h)),
            full((1, d_h)),
            full((1, 1)),
        ],
        out_specs=[
            pl.BlockSpec((blk, 1), lambda i: (i, 0)),
            pl.BlockSpec((blk, d_h), lambda i: (i, 0)),
            pl.BlockSpec((blk, d_h), lambda i: (i, 0)),
        ],
        out_shape=[
            jax.ShapeDtypeStruct((n, 1), jnp.float32),
            jax.ShapeDtypeStruct((n, d_h), jnp.float32),
            jax.ShapeDtypeStruct((n, d_h), jnp.float32),
        ],
    )(acc, x, h, c, wx, wxr, wh, whr, bias, fcw, fcb)


def kernel(x, edge_index, edge_weight, h, c,
           Wxi, Wxi_root, bxi, Whi, Whi_root, bhi,
           Wxf, Wxf_root, bxf, Whf, Whf_root, bhf,
           Wxc, Wxc_root, bxc, Whc, Whc_root, bhc,
           Wxo, Wxo_root, bxo, Who, Who_root, bho,
           fc_w, fc_b):
    n, d_in = x.shape
    d_h = h.shape[1]
    e = edge_index.shape[1]

    e_pad = NS * NG * GG * CK
    r_acc = 8 * (-(-(n + 1) // 8))
    d_full = 16 * (-(-(d_in + d_h + 1) // 16))
    d_t = d_full - SPLIT

    src = jnp.concatenate(
        [edge_index[0], jnp.full((e_pad - e,), n, jnp.int32)])
    dst = jnp.concatenate(
        [edge_index[1], jnp.full((e_pad - e,), n, jnp.int32)])
    idx = jnp.stack([src.reshape(NS, NG, GG, CK),
                     dst.reshape(NS, NG, GG, CK)], axis=3)

    tbl = jnp.zeros((NC, r_acc, d_t), jnp.float32)
    tbl = tbl.at[0, :n, :SPLIT].set(x[:, :SPLIT])
    tbl = tbl.at[1, :n, :d_in - SPLIT].set(x[:, SPLIT:])
    tbl = tbl.at[1, :n, d_in - SPLIT:d_in - SPLIT + d_h].set(h)
    tbl = tbl.at[1, :n, d_in - SPLIT + d_h].set(1.0)
    zro = jnp.zeros((r_acc, d_t), jnp.float32)

    acc = _sc_segment_accumulate(tbl, idx, zro, r_acc=r_acc, d_t=d_t)

    wx = jnp.concatenate([Wxi, Wxf, Wxc, Wxo], axis=1)
    wxr = jnp.concatenate([Wxi_root, Wxf_root, Wxc_root, Wxo_root], axis=1)
    wh = jnp.concatenate([Whi, Whf, Whc, Who], axis=1)
    whr = jnp.concatenate([Whi_root, Whf_root, Whc_root, Who_root], axis=1)
    bias = jnp.concatenate(
        [bxi + bhi, bxf + bhf, bxc + bhc, bxo + bho])[None, :]

    out, h_new, c_new = _tc_gates(acc, x, h, c, wx, wxr, wh, whr, bias,
                                  fc_w, fc_b.reshape(1, 1),
                                  n=n, d_in=d_in, d_h=d_h)
    return (out, h_new, c_new)


# speedup vs baseline: 1.2806x; 1.2806x over previous
import functools

import jax
import jax.numpy as jnp
from jax import lax
from jax.experimental import pallas as pl
from jax.experimental.pallas import tpu as pltpu
from jax.experimental.pallas import tpu_sc as plsc

NC = 2
NS = 16
CK = 128
SPLIT = 104
GG = 18
NG = 9
NR = 3


def _sc_segment_accumulate(tbl, idx, zro, *, r_acc, d_t):
    rows_main = 8 * (-(-r_acc // (NS * 8)))
    rows_last = r_acc - (NS - 1) * rows_main
    assert rows_last > 0
    mesh = plsc.VectorSubcoreMesh(core_axis_name="c", subcore_axis_name="s")

    @functools.partial(
        pl.kernel,
        out_type=jax.ShapeDtypeStruct((NC, r_acc, d_t), jnp.float32),
        mesh=mesh,
        scratch_types=[
            pltpu.VMEM((GG, 2, CK), jnp.int32),
            pltpu.VMEM((GG, 2, CK), jnp.int32),
            pltpu.VMEM((CK, d_t), jnp.float32),
            pltpu.VMEM((CK, d_t), jnp.float32),
            pltpu.VMEM((CK, d_t), jnp.float32),
            pltpu.VMEM_SHARED((r_acc, d_t), jnp.float32),
            pltpu.SemaphoreType.DMA,
            pltpu.SemaphoreType.DMA,
            pltpu.SemaphoreType.DMA,
            pltpu.SemaphoreType.DMA,
            pltpu.SemaphoreType.DMA,
        ],
        compiler_params=pltpu.CompilerParams(use_tc_tiling_on_sc=False),
    )
    def body(tbl_hbm, idx_hbm, zro_hbm, acc_out, idx_a, idx_b,
             rows0, rows1, rows2, acc_sh, semi_a, semi_b, sem0, sem1, sem2):
        cid = lax.axis_index("c")
        sid = lax.axis_index("s")
        rbase = pl.multiple_of(sid * rows_main, 8)

        @pl.when(sid < NS - 1)
        def _init_main():
            pltpu.sync_copy(zro_hbm.at[pl.ds(rbase, rows_main)],
                            acc_sh.at[pl.ds(rbase, rows_main)])

        @pl.when(sid == NS - 1)
        def _init_last():
            pltpu.sync_copy(zro_hbm.at[pl.ds((NS - 1) * rows_main, rows_last)],
                            acc_sh.at[pl.ds((NS - 1) * rows_main, rows_last)])

        plsc.subcore_barrier()

        rows = (rows0, rows1, rows2)
        gsems = (sem0, sem1, sem2)
        idxs = (idx_a, idx_b)
        isems = (semi_a, semi_b)

        def start_gather(idx_v, j, b):
            pltpu.async_copy(tbl_hbm.at[cid].at[idx_v.at[j, 0]],
                             rows[b], gsems[b])

        def wait_gather(b):
            pltpu.make_async_copy(tbl_hbm.at[cid, pl.ds(0, CK)],
                                  rows[b], gsems[b]).wait()

        def scatter(idx_v, j, b):
            pltpu.sync_copy(rows[b], acc_sh.at[idx_v.at[j, 1]], add=True)

        pltpu.sync_copy(idx_hbm.at[sid, 0], idx_a)

        for p in range(NG):
            cur, nxt = idxs[p % 2], idxs[(p + 1) % 2]
            if p + 1 < NG:
                pltpu.async_copy(idx_hbm.at[sid, p + 1], nxt,
                                 isems[(p + 1) % 2])
            start_gather(cur, 0, 0)
            start_gather(cur, 1, 1)

            @pl.loop(0, GG, step=NR)
            def _step(j, cur=cur):
                start_gather(cur, j + 2, 2)
                wait_gather(0)
                scatter(cur, j, 0)

                @pl.when(j + 3 < GG)
                def _r0(cur=cur, j=j):
                    start_gather(cur, j + 3, 0)

                wait_gather(1)
                scatter(cur, j + 1, 1)

                @pl.when(j + 4 < GG)
                def _r1(cur=cur, j=j):
                    start_gather(cur, j + 4, 1)

                wait_gather(2)
                scatter(cur, j + 2, 2)

            if p + 1 < NG:
                pltpu.make_async_copy(idx_hbm.at[sid, 0], nxt,
                                      isems[(p + 1) % 2]).wait()

        plsc.subcore_barrier()

        @pl.when(sid < NS - 1)
        def _wb_main():
            pltpu.sync_copy(acc_sh.at[pl.ds(rbase, rows_main)],
                            acc_out.at[cid, pl.ds(rbase, rows_main)])

        @pl.when(sid == NS - 1)
        def _wb_last():
            pltpu.sync_copy(acc_sh.at[pl.ds((NS - 1) * rows_main, rows_last)],
                            acc_out.at[cid, pl.ds((NS - 1) * rows_main,
                                                  rows_last)])

    return body(tbl, idx, zro)


def _tc_gates(acc, x, h, c, wx, wxr, wh, whr, bias, fcw, fcb, *, n, d_in, d_h):
    d_t = acc.shape[-1]
    blk = 2000
    grid = (n // blk,)

    def body(acc_ref, x_ref, h_ref, c_ref, wx_ref, wxr_ref, wh_ref, whr_ref,
             b_ref, fcw_ref, fcb_ref, out_ref, hn_ref, cn_ref):
        s = jnp.concatenate([acc_ref[0, :, :SPLIT], acc_ref[1]], axis=-1)
        deg = jnp.maximum(s[:, d_in + d_h:d_in + d_h + 1], 1.0)
        inv = 1.0 / deg
        zx = s[:, :d_in] * inv
        zh = s[:, d_in:d_in + d_h] * inv
        pre = (jnp.dot(zx, wx_ref[...], preferred_element_type=jnp.float32)
               + jnp.dot(x_ref[...], wxr_ref[...],
                         preferred_element_type=jnp.float32)
               + jnp.dot(zh, wh_ref[...], preferred_element_type=jnp.float32)
               + jnp.dot(h_ref[...], whr_ref[...],
                         preferred_element_type=jnp.float32)
               + b_ref[...])
        ig = jax.nn.sigmoid(pre[:, :d_h])
        fg = jax.nn.sigmoid(pre[:, d_h:2 * d_h])
        tg = jnp.tanh(pre[:, 2 * d_h:3 * d_h])
        og = jax.nn.sigmoid(pre[:, 3 * d_h:])
        cn = fg * c_ref[...] + ig * tg
        hn = og * jnp.tanh(cn)
        r = jnp.maximum(hn, 0.0)
        out_ref[...] = (jnp.sum(r * fcw_ref[...], axis=1, keepdims=True)
                        + fcb_ref[0, 0])
        hn_ref[...] = hn
        cn_ref[...] = cn

    full = lambda shape: pl.BlockSpec(shape, lambda i: tuple(0 for _ in shape))
    return pl.pallas_call(
        body,
        grid=grid,
        in_specs=[
            pl.BlockSpec((NC, blk, d_t), lambda i: (0, i, 0)),
            pl.BlockSpec((blk, d_in), lambda i: (i, 0)),
            pl.BlockSpec((blk, d_h), lambda i: (i, 0)),
            pl.BlockSpec((blk, d_h), lambda i: (i, 0)),
            full((d_in, 4 * d_h)),
            full((d_in, 4 * d_h)),
            full((d_h, 4 * d_h)),
            full((d_h, 4 * d_h)),
            full((1, 4 * d_h)),
            full((1, d_h)),
            full((1, 1)),
        ],
        out_specs=[
            pl.BlockSpec((blk, 1), lambda i: (i, 0)),
            pl.BlockSpec((blk, d_h), lambda i: (i, 0)),
            pl.BlockSpec((blk, d_h), lambda i: (i, 0)),
        ],
        out_shape=[
            jax.ShapeDtypeStruct((n, 1), jnp.float32),
            jax.ShapeDtypeStruct((n, d_h), jnp.float32),
            jax.ShapeDtypeStruct((n, d_h), jnp.float32),
        ],
    )(acc, x, h, c, wx, wxr, wh, whr, bias, fcw, fcb)


def kernel(x, edge_index, edge_weight, h, c,
           Wxi, Wxi_root, bxi, Whi, Whi_root, bhi,
           Wxf, Wxf_root, bxf, Whf, Whf_root, bhf,
           Wxc, Wxc_root, bxc, Whc, Whc_root, bhc,
           Wxo, Wxo_root, bxo, Who, Who_root, bho,
           fc_w, fc_b):
    n, d_in = x.shape
    d_h = h.shape[1]
    e = edge_index.shape[1]

    e_pad = NS * NG * GG * CK
    r_acc = 8 * (-(-(n + 1) // 8))
    d_full = 16 * (-(-(d_in + d_h + 1) // 16))
    d_t = d_full - SPLIT

    src = jnp.concatenate(
        [edge_index[0], jnp.full((e_pad - e,), n, jnp.int32)])
    dst = jnp.concatenate(
        [edge_index[1], jnp.full((e_pad - e,), n, jnp.int32)])
    idx = jnp.stack([src.reshape(NS, NG, GG, CK),
                     dst.reshape(NS, NG, GG, CK)], axis=3)

    tbl = jnp.zeros((NC, r_acc, d_t), jnp.float32)
    tbl = tbl.at[0, :n, :SPLIT].set(x[:, :SPLIT])
    tbl = tbl.at[1, :n, :d_in - SPLIT].set(x[:, SPLIT:])
    tbl = tbl.at[1, :n, d_in - SPLIT:d_in - SPLIT + d_h].set(h)
    tbl = tbl.at[1, :n, d_in - SPLIT + d_h].set(1.0)
    zro = jnp.zeros((r_acc, d_t), jnp.float32)

    acc = _sc_segment_accumulate(tbl, idx, zro, r_acc=r_acc, d_t=d_t)

    wx = jnp.concatenate([Wxi, Wxf, Wxc, Wxo], axis=1)
    wxr = jnp.concatenate([Wxi_root, Wxf_root, Wxc_root, Wxo_root], axis=1)
    wh = jnp.concatenate([Whi, Whf, Whc, Who], axis=1)
    whr = jnp.concatenate([Whi_root, Whf_root, Whc_root, Who_root], axis=1)
    bias = jnp.concatenate(
        [bxi + bhi, bxf + bhf, bxc + bhc, bxo + bho])[None, :]

    out, h_new, c_new = _tc_gates(acc, x, h, c, wx, wxr, wh, whr, bias,
                                  fc_w, fc_b.reshape(1, 1),
                                  n=n, d_in=d_in, d_h=d_h)
    return (out, h_new, c_new)


# speedup vs baseline: 1.5709x; 1.2267x over previous
import functools

import jax
import jax.numpy as jnp
from jax import lax
from jax.experimental import pallas as pl
from jax.experimental.pallas import tpu as pltpu
from jax.experimental.pallas import tpu_sc as plsc

NC = 2
NS = 16
CK = 128
SPLIT = 104
GG = 16
NG = 10
NR = 4


def _sc_segment_accumulate(tbl, idx, zro, *, r_acc, d_t):
    rows_main = 8 * (-(-r_acc // (NS * 8)))
    rows_last = r_acc - (NS - 1) * rows_main
    assert rows_last > 0
    mesh = plsc.VectorSubcoreMesh(core_axis_name="c", subcore_axis_name="s")

    @functools.partial(
        pl.kernel,
        out_type=jax.ShapeDtypeStruct((NC, r_acc, d_t), jnp.float32),
        mesh=mesh,
        scratch_types=[
            pltpu.VMEM((GG, 2, CK), jnp.int32),
            pltpu.VMEM((GG, 2, CK), jnp.int32),
            pltpu.VMEM((CK, d_t), jnp.float32),
            pltpu.VMEM((CK, d_t), jnp.float32),
            pltpu.VMEM((CK, d_t), jnp.float32),
            pltpu.VMEM((CK, d_t), jnp.float32),
            pltpu.VMEM_SHARED((r_acc, d_t), jnp.float32),
            pltpu.SemaphoreType.DMA,
            pltpu.SemaphoreType.DMA,
            pltpu.SemaphoreType.DMA,
            pltpu.SemaphoreType.DMA,
            pltpu.SemaphoreType.DMA,
            pltpu.SemaphoreType.DMA,
        ],
        compiler_params=pltpu.CompilerParams(use_tc_tiling_on_sc=False),
    )
    def body(tbl_hbm, idx_hbm, zro_hbm, acc_out, idx_a, idx_b,
             rows0, rows1, rows2, rows3, acc_sh, semi_a, semi_b,
             sem0, sem1, sem2, sem3):
        cid = lax.axis_index("c")
        sid = lax.axis_index("s")
        rbase = pl.multiple_of(sid * rows_main, 8)

        @pl.when(sid < NS - 1)
        def _init_main():
            pltpu.sync_copy(zro_hbm.at[pl.ds(rbase, rows_main)],
                            acc_sh.at[pl.ds(rbase, rows_main)])

        @pl.when(sid == NS - 1)
        def _init_last():
            pltpu.sync_copy(zro_hbm.at[pl.ds((NS - 1) * rows_main, rows_last)],
                            acc_sh.at[pl.ds((NS - 1) * rows_main, rows_last)])

        plsc.subcore_barrier()

        rows = (rows0, rows1, rows2, rows3)
        gsems = (sem0, sem1, sem2, sem3)
        idxs = (idx_a, idx_b)
        isems = (semi_a, semi_b)

        def start_gather(idx_v, j, b):
            pltpu.async_copy(tbl_hbm.at[cid].at[idx_v.at[j, 0]],
                             rows[b], gsems[b])

        def wait_gather(b):
            pltpu.make_async_copy(tbl_hbm.at[cid, pl.ds(0, CK)],
                                  rows[b], gsems[b]).wait()

        def scatter(idx_v, j, b):
            pltpu.sync_copy(rows[b], acc_sh.at[idx_v.at[j, 1]], add=True)

        pltpu.sync_copy(idx_hbm.at[sid, 0], idx_a)

        for p in range(NG):
            cur, nxt = idxs[p % 2], idxs[(p + 1) % 2]
            if p + 1 < NG:
                pltpu.async_copy(idx_hbm.at[sid, p + 1], nxt,
                                 isems[(p + 1) % 2])
            for b in range(NR - 1):
                start_gather(cur, b, b)

            @pl.loop(0, GG, step=NR)
            def _step(j, cur=cur):
                start_gather(cur, j + NR - 1, NR - 1)
                for k in range(NR):
                    wait_gather(k)
                    scatter(cur, j + k, k)
                    if k < NR - 1:
                        @pl.when(j + k + NR < GG)
                        def _refill(cur=cur, j=j, k=k):
                            start_gather(cur, j + k + NR, k)

            if p + 1 < NG:
                pltpu.make_async_copy(idx_hbm.at[sid, 0], nxt,
                                      isems[(p + 1) % 2]).wait()

        plsc.subcore_barrier()

        @pl.when(sid < NS - 1)
        def _wb_main():
            pltpu.sync_copy(acc_sh.at[pl.ds(rbase, rows_main)],
                            acc_out.at[cid, pl.ds(rbase, rows_main)])

        @pl.when(sid == NS - 1)
        def _wb_last():
            pltpu.sync_copy(acc_sh.at[pl.ds((NS - 1) * rows_main, rows_last)],
                            acc_out.at[cid, pl.ds((NS - 1) * rows_main,
                                                  rows_last)])

    return body(tbl, idx, zro)


def _tc_gates(acc, x, h, c, wx, wxr, wh, whr, bias, fcw, fcb, *, n, d_in, d_h):
    d_t = acc.shape[-1]
    blk = 2000
    grid = (n // blk,)

    def body(acc_ref, x_ref, h_ref, c_ref, wx_ref, wxr_ref, wh_ref, whr_ref,
             b_ref, fcw_ref, fcb_ref, out_ref, hn_ref, cn_ref):
        s = jnp.concatenate([acc_ref[0, :, :SPLIT], acc_ref[1]], axis=-1)
        deg = jnp.maximum(s[:, d_in + d_h:d_in + d_h + 1], 1.0)
        inv = 1.0 / deg
        zx = s[:, :d_in] * inv
        zh = s[:, d_in:d_in + d_h] * inv
        pre = (jnp.dot(zx, wx_ref[...], preferred_element_type=jnp.float32)
               + jnp.dot(x_ref[...], wxr_ref[...],
                         preferred_element_type=jnp.float32)
               + jnp.dot(zh, wh_ref[...], preferred_element_type=jnp.float32)
               + jnp.dot(h_ref[...], whr_ref[...],
                         preferred_element_type=jnp.float32)
               + b_ref[...])
        ig = jax.nn.sigmoid(pre[:, :d_h])
        fg = jax.nn.sigmoid(pre[:, d_h:2 * d_h])
        tg = jnp.tanh(pre[:, 2 * d_h:3 * d_h])
        og = jax.nn.sigmoid(pre[:, 3 * d_h:])
        cn = fg * c_ref[...] + ig * tg
        hn = og * jnp.tanh(cn)
        r = jnp.maximum(hn, 0.0)
        out_ref[...] = (jnp.sum(r * fcw_ref[...], axis=1, keepdims=True)
                        + fcb_ref[0, 0])
        hn_ref[...] = hn
        cn_ref[...] = cn

    full = lambda shape: pl.BlockSpec(shape, lambda i: tuple(0 for _ in shape))
    return pl.pallas_call(
        body,
        grid=grid,
        in_specs=[
            pl.BlockSpec((NC, blk, d_t), lambda i: (0, i, 0)),
            pl.BlockSpec((blk, d_in), lambda i: (i, 0)),
            pl.BlockSpec((blk, d_h), lambda i: (i, 0)),
            pl.BlockSpec((blk, d_h), lambda i: (i, 0)),
            full((d_in, 4 * d_h)),
            full((d_in, 4 * d_h)),
            full((d_h, 4 * d_h)),
            full((d_h, 4 * d_h)),
            full((1, 4 * d_h)),
            full((1, d_h)),
            full((1, 1)),
        ],
        out_specs=[
            pl.BlockSpec((blk, 1), lambda i: (i, 0)),
            pl.BlockSpec((blk, d_h), lambda i: (i, 0)),
            pl.BlockSpec((blk, d_h), lambda i: (i, 0)),
        ],
        out_shape=[
            jax.ShapeDtypeStruct((n, 1), jnp.float32),
            jax.ShapeDtypeStruct((n, d_h), jnp.float32),
            jax.ShapeDtypeStruct((n, d_h), jnp.float32),
        ],
    )(acc, x, h, c, wx, wxr, wh, whr, bias, fcw, fcb)


def kernel(x, edge_index, edge_weight, h, c,
           Wxi, Wxi_root, bxi, Whi, Whi_root, bhi,
           Wxf, Wxf_root, bxf, Whf, Whf_root, bhf,
           Wxc, Wxc_root, bxc, Whc, Whc_root, bhc,
           Wxo, Wxo_root, bxo, Who, Who_root, bho,
           fc_w, fc_b):
    n, d_in = x.shape
    d_h = h.shape[1]
    e = edge_index.shape[1]

    e_pad = NS * NG * GG * CK
    r_acc = 8 * (-(-(n + 1) // 8))
    d_full = 16 * (-(-(d_in + d_h + 1) // 16))
    d_t = d_full - SPLIT

    src = jnp.concatenate(
        [edge_index[0], jnp.full((e_pad - e,), n, jnp.int32)])
    dst = jnp.concatenate(
        [edge_index[1], jnp.full((e_pad - e,), n, jnp.int32)])
    idx = jnp.stack([src.reshape(NS, NG, GG, CK),
                     dst.reshape(NS, NG, GG, CK)], axis=3)

    tbl = jnp.zeros((NC, r_acc, d_t), jnp.float32)
    tbl = tbl.at[0, :n, :SPLIT].set(x[:, :SPLIT])
    tbl = tbl.at[1, :n, :d_in - SPLIT].set(x[:, SPLIT:])
    tbl = tbl.at[1, :n, d_in - SPLIT:d_in - SPLIT + d_h].set(h)
    tbl = tbl.at[1, :n, d_in - SPLIT + d_h].set(1.0)
    zro = jnp.zeros((r_acc, d_t), jnp.float32)

    acc = _sc_segment_accumulate(tbl, idx, zro, r_acc=r_acc, d_t=d_t)

    wx = jnp.concatenate([Wxi, Wxf, Wxc, Wxo], axis=1)
    wxr = jnp.concatenate([Wxi_root, Wxf_root, Wxc_root, Wxo_root], axis=1)
    wh = jnp.concatenate([Whi, Whf, Whc, Who], axis=1)
    whr = jnp.concatenate([Whi_root, Whf_root, Whc_root, Who_root], axis=1)
    bias = jnp.concatenate(
        [bxi + bhi, bxf + bhf, bxc + bhc, bxo + bho])[None, :]

    out, h_new, c_new = _tc_gates(acc, x, h, c, wx, wxr, wh, whr, bias,
                                  fc_w, fc_b.reshape(1, 1),
                                  n=n, d_in=d_in, d_h=d_h)
    return (out, h_new, c_new)
